# baseline (speedup 1.0000x reference)
import functools

import jax
import jax.numpy as jnp
import numpy as np
from jax import lax
from jax.experimental import pallas as pl
from jax.experimental.pallas import tpu as pltpu
from jax.experimental.pallas import tpu_sc as plsc

N_NODES = 10000
N_EDGES = 320000
D = 128
DH = 64

NC = 2
NS = 16
NW = NC * NS

N_PAD = 10240
ECHUNK = 128
CPT = 160
EPT = CPT * ECHUNK
E_PAD = NS * EPT
PAD_IDX = N_PAD - 1

NBUF = 4
LOOKAHEAD = 2
BLK = 8
RB = 3
RROWS = RB * BLK
NBLK = CPT // BLK
TROWS = NS * CPT
SLAB_ROWS = TROWS + 2 * BLK

ROWS_PER_TILE = N_PAD // NS
DEG_PER_TILE = E_PAD // NS


def _deg_body(srcw_hbm, dstw_hbm, deg_hbm, slab_v, deg2d_v, iota_v, zbuf_v, acc):
  c = lax.axis_index("c")
  s = lax.axis_index("s")

  @pl.when(c == 0)
  def _():
    pltpu.sync_copy(srcw_hbm.at[pl.ds(s * CPT, CPT)], slab_v)

  @pl.when(c == 1)
  def _():
    pltpu.sync_copy(dstw_hbm.at[pl.ds(s * CPT, CPT)], slab_v)

  for k in range(8):
    z = jnp.zeros((16,), jnp.float32)
    for r in range(8):
      zbuf_v[r, pl.ds(16 * k, 16)] = z

  @pl.when(s < 10)
  def _():
    pltpu.sync_copy(zbuf_v, acc.at[pl.ds(s * 8, 8)])

  @pl.loop(0, 80)
  def _(r):
    z16 = jnp.zeros((16,), jnp.float32)
    for k in range(8):
      deg2d_v[r, pl.ds(16 * k, 16)] = z16

  for r in range(5):
    iota_v[r, :] = lax.iota(jnp.int32, 16) + 16 * r
  plsc.subcore_barrier()

  ones = jnp.ones((16,), jnp.float32)

  @pl.loop(0, CPT)
  def _(r):
    for k in range(ECHUNK // 16):
      v = slab_v[r, pl.ds(16 * k, 16)]
      rows = lax.shift_right_logical(v, 7)
      cols = jnp.bitwise_and(v, 127)
      plsc.addupdate_scatter(deg2d_v, (rows, cols), ones)

  for r in range(5):
    pltpu.sync_copy(deg2d_v.at[pl.ds(r * 16, 16)], acc.at[iota_v.at[r]],
                    add=True)
  plsc.subcore_barrier()

  @pl.when(s < 10)
  def _():
    pltpu.sync_copy(acc.at[pl.ds(s * 8, 8)], deg_hbm.at[c, pl.ds(s * 8, 8)])


def _degrees(srcw, dstw):
  mesh = plsc.VectorSubcoreMesh(core_axis_name="c", subcore_axis_name="s",
                                num_cores=NC, num_subcores=NS)
  return pl.kernel(
      _deg_body,
      out_type=jax.ShapeDtypeStruct((2, 80, 128), jnp.float32),
      mesh=mesh,
      compiler_params=pltpu.CompilerParams(needs_layout_passes=False),
      scratch_types=[
          pltpu.VMEM((CPT, ECHUNK), jnp.int32),
          pltpu.VMEM((80, 128), jnp.float32),
          pltpu.VMEM((5, 16), jnp.int32),
          pltpu.VMEM((8, 128), jnp.float32),
          pltpu.VMEM_SHARED((80, 128), jnp.float32),
      ],
  )(srcw, dstw)


def _prop_body(z_hbm, srcw_hbm, dstw_hbm, p_hbm,
               ring_s, ring_d, *rest):
  bufs = rest[:NBUF]
  zbuf_v = rest[NBUF]
  gsems = rest[NBUF + 1:2 * NBUF + 1]
  ssems = rest[2 * NBUF + 1:3 * NBUF + 1]
  rsem = rest[3 * NBUF + 1]
  zsp = rest[3 * NBUF + 2]
  acc = rest[3 * NBUF + 3]

  c = lax.axis_index("c")
  s = lax.axis_index("s")

  pltpu.sync_copy(z_hbm.at[pl.ds(c * N_PAD + s * ROWS_PER_TILE, ROWS_PER_TILE)],
                  zsp.at[pl.ds(s * ROWS_PER_TILE, ROWS_PER_TILE)])

  for k in range(4):
    z = jnp.zeros((16,), jnp.float32)
    for r in range(16):
      zbuf_v[r, pl.ds(16 * k, 16)] = z

  @pl.loop(0, ROWS_PER_TILE // 16)
  def _(i):
    pltpu.async_copy(zbuf_v, acc.at[pl.ds(s * ROWS_PER_TILE + i * 16, 16)],
                     rsem)

  @pl.loop(0, ROWS_PER_TILE // 16)
  def _(i):
    pltpu.make_async_copy(
        zbuf_v, acc.at[pl.ds(s * ROWS_PER_TILE, 16)], rsem).wait()

  def slab_row(r0):
    if isinstance(r0, int):
      return s * CPT + r0 if r0 < CPT else TROWS + (r0 - CPT)
    return jnp.where(r0 < CPT, s * CPT + r0, TROWS + (r0 - CPT))

  def issue_ring(row0):
    rr = row0 % RROWS if isinstance(row0, int) else lax.rem(row0, RROWS)
    g0 = slab_row(row0)
    pltpu.async_copy(srcw_hbm.at[pl.ds(g0, BLK)],
                     ring_s.at[pl.ds(rr, BLK)], rsem)
    pltpu.async_copy(dstw_hbm.at[pl.ds(g0, BLK)],
                     ring_d.at[pl.ds(rr, BLK)], rsem)

  def wait_ring():
    pltpu.make_async_copy(srcw_hbm.at[pl.ds(0, BLK)],
                          ring_s.at[pl.ds(0, BLK)], rsem).wait()
    pltpu.make_async_copy(dstw_hbm.at[pl.ds(0, BLK)],
                          ring_d.at[pl.ds(0, BLK)], rsem).wait()

  def issue_gather(chunk, b):
    rr = chunk % RROWS if isinstance(chunk, int) else lax.rem(chunk, RROWS)
    pltpu.async_copy(zsp.at[ring_s.at[rr]], bufs[b], gsems[b])

  def wait_gather(b):
    pltpu.make_async_copy(zsp.at[ring_s.at[0]], bufs[b], gsems[b]).wait()

  def issue_scatter(chunk, b):
    rr = chunk % RROWS if isinstance(chunk, int) else lax.rem(chunk, RROWS)
    pltpu.async_copy(bufs[b], acc.at[ring_d.at[rr]], ssems[b], add=True)

  def wait_scatter(b):
    pltpu.make_async_copy(bufs[b], acc.at[ring_d.at[0]], ssems[b]).wait()

  for g in range(RB):
    issue_ring(g * BLK)
  wait_ring()
  wait_ring()

  plsc.subcore_barrier()

  for i in range(LOOKAHEAD):
    issue_gather(i, i % NBUF)
  for i in range(BLK):
    bf = (i + LOOKAHEAD) % NBUF
    if i + LOOKAHEAD >= NBUF:
      wait_scatter(bf)
    issue_gather(i + LOOKAHEAD, bf)
    wait_gather(i % NBUF)
    issue_scatter(i, i % NBUF)

  @pl.loop(BLK, CPT, step=BLK)
  def _(j):
    wait_ring()
    for p in range(BLK):
      i = j + p
      b = p % NBUF
      bf = (p + LOOKAHEAD) % NBUF
      wait_scatter(bf)
      issue_gather(i + LOOKAHEAD, bf)
      wait_gather(b)
      issue_scatter(i, b)
    issue_ring(j + 2 * BLK)

  for i in range(CPT - (NBUF - LOOKAHEAD), CPT):
    wait_scatter(i % NBUF)
  for i in range(CPT, CPT + LOOKAHEAD):
    wait_gather(i % NBUF)
  wait_ring()

  plsc.subcore_barrier()
  pltpu.sync_copy(acc.at[pl.ds(s * ROWS_PER_TILE, ROWS_PER_TILE)],
                  p_hbm.at[c, pl.ds(s * ROWS_PER_TILE, ROWS_PER_TILE)])


@functools.cache
def _propagate_kernel():
  mesh = plsc.VectorSubcoreMesh(core_axis_name="c", subcore_axis_name="s",
                                num_cores=NC, num_subcores=NS)
  return pl.kernel(
      _prop_body,
      out_type=jax.ShapeDtypeStruct((2, N_PAD, DH), jnp.float32),
      mesh=mesh,
      compiler_params=pltpu.CompilerParams(needs_layout_passes=False,
                                           use_tc_tiling_on_sc=False),
      scratch_types=(
          [pltpu.VMEM((RROWS, ECHUNK), jnp.int32),
           pltpu.VMEM((RROWS, ECHUNK), jnp.int32)]
          + [pltpu.VMEM((ECHUNK, DH), jnp.float32) for _ in range(NBUF)]
          + [pltpu.VMEM((16, DH), jnp.float32)]
          + [pltpu.SemaphoreType.DMA for _ in range(2 * NBUF + 1)]
          + [pltpu.VMEM_SHARED((N_PAD, DH), jnp.float32),
             pltpu.VMEM_SHARED((N_PAD, DH), jnp.float32)]
      ),
  )


@jax.jit
def _propagate(zflat, srcw, dstw):
  return _propagate_kernel()(zflat, srcw, dstw)


def _norm_col(deg_ref):
  return lax.rsqrt(jnp.maximum(deg_ref[...], 1.0))


def _emit(e2, o2, th):
  return jnp.concatenate(
      [jnp.dot(e2, th, preferred_element_type=jnp.float32),
       jnp.dot(o2, th, preferred_element_type=jnp.float32)], axis=1)


def _deinter(et2, ot2, q):
  return (jnp.dot(et2, q[:, :64], preferred_element_type=jnp.float32)
          + jnp.dot(ot2, q[:, 64:], preferred_element_type=jnp.float32))


def _mm1_body(degs_ref, x_ref, w_ref, e2_ref, o2_ref, o_ref):
  t = jnp.dot(x_ref[...], w_ref[...], preferred_element_type=jnp.float32)
  ts = t * _norm_col(degs_ref)
  e2, o2 = e2_ref[...], o2_ref[...]
  q0 = _emit(e2, o2, ts[:, :64])
  q1 = _emit(e2, o2, ts[:, 64:])
  o_ref[...] = jnp.concatenate([q0[None], q1[None]], axis=0)


def _mm1(degs_col, x_pad, w1, e2, o2):
  return pl.pallas_call(
      _mm1_body,
      grid=(10,),
      in_specs=[
          pl.BlockSpec((1024, 1), lambda i: (i, 0)),
          pl.BlockSpec((1024, 128), lambda i: (i, 0)),
          pl.BlockSpec((128, 128), lambda i: (0, 0)),
          pl.BlockSpec((512, 1024), lambda i: (0, 0)),
          pl.BlockSpec((512, 1024), lambda i: (0, 0)),
      ],
      out_specs=pl.BlockSpec((2, 512, 128), lambda i: (0, i, 0)),
      out_shape=jax.ShapeDtypeStruct((2, N_PAD // 2, 128), jnp.float32),
  )(degs_col, x_pad, w1, e2, o2)


def _mm2_body(degs_ref, degd_ref, p_ref, w_ref, b_ref, e2_ref, o2_ref,
              et2_ref, ot2_ref, o_ref):
  et2, ot2 = et2_ref[...], ot2_ref[...]
  cat = jnp.concatenate([_deinter(et2, ot2, p_ref[0]),
                         _deinter(et2, ot2, p_ref[1])], axis=1)
  h = cat * _norm_col(degd_ref) + b_ref[...]
  t = jnp.dot(h, w_ref[...], preferred_element_type=jnp.float32)
  ts = t * _norm_col(degs_ref)
  e2, o2 = e2_ref[...], o2_ref[...]
  q0 = _emit(e2, o2, ts[:, :64])
  q1 = _emit(e2, o2, ts[:, 64:])
  o_ref[...] = jnp.concatenate([q0[None], q1[None]], axis=0)


def _mm2(degs_col, degd_col, p, w2, b1, e2, o2, et2, ot2):
  return pl.pallas_call(
      _mm2_body,
      grid=(10,),
      in_specs=[
          pl.BlockSpec((1024, 1), lambda i: (i, 0)),
          pl.BlockSpec((1024, 1), lambda i: (i, 0)),
          pl.BlockSpec((2, 512, 128), lambda i: (0, i, 0)),
          pl.BlockSpec((128, 128), lambda i: (0, 0)),
          pl.BlockSpec((1, 128), lambda i: (0, 0)),
          pl.BlockSpec((512, 1024), lambda i: (0, 0)),
          pl.BlockSpec((512, 1024), lambda i: (0, 0)),
          pl.BlockSpec((1024, 512), lambda i: (0, 0)),
          pl.BlockSpec((1024, 512), lambda i: (0, 0)),
      ],
      out_specs=pl.BlockSpec((2, 512, 128), lambda i: (0, i, 0)),
      out_shape=jax.ShapeDtypeStruct((2, N_PAD // 2, 128), jnp.float32),
  )(degs_col, degd_col, p, w2, b1, e2, o2, et2, ot2)


def _fin_body(degd_ref, p_ref, b_ref, et2_ref, ot2_ref, o_ref):
  et2, ot2 = et2_ref[...], ot2_ref[...]
  cat = jnp.concatenate([_deinter(et2, ot2, p_ref[0]),
                         _deinter(et2, ot2, p_ref[1])], axis=1)
  o_ref[...] = cat * _norm_col(degd_ref) + b_ref[...]


def _fin(degd_col, p, b2, et2, ot2):
  return pl.pallas_call(
      _fin_body,
      grid=(10,),
      in_specs=[
          pl.BlockSpec((1024, 1), lambda i: (i, 0)),
          pl.BlockSpec((2, 512, 128), lambda i: (0, i, 0)),
          pl.BlockSpec((1, 128), lambda i: (0, 0)),
          pl.BlockSpec((1024, 512), lambda i: (0, 0)),
          pl.BlockSpec((1024, 512), lambda i: (0, 0)),
      ],
      out_specs=pl.BlockSpec((1024, 128), lambda i: (i, 0)),
      out_shape=jax.ShapeDtypeStruct((N_PAD, D), jnp.float32),
  )(degd_col, p, b2, et2, ot2)


@jax.jit
def kernel(x, edge_index, W1, b1, W2, b2):
  src = edge_index[0].astype(jnp.int32)
  dst = edge_index[1].astype(jnp.int32)
  pad = jnp.full((SLAB_ROWS * ECHUNK - N_EDGES,), PAD_IDX, jnp.int32)
  srcw = jnp.concatenate([src, pad]).reshape(SLAB_ROWS, ECHUNK)
  dstw = jnp.concatenate([dst, pad]).reshape(SLAB_ROWS, ECHUNK)

  deg = _degrees(srcw, dstw)
  degs_col = deg[0].reshape(N_PAD, 1)
  degd_col = deg[1].reshape(N_PAD, 1)

  x_pad = jnp.pad(x, ((0, N_PAD - N_NODES), (0, 0)))
  b1r = b1.reshape(1, D)
  b2r = b2.reshape(1, D)

  ii = np.arange(512)
  e2_np = np.zeros((512, 1024), np.float32)
  o2_np = np.zeros((512, 1024), np.float32)
  e2_np[ii, 2 * ii] = 1.0
  o2_np[ii, 2 * ii + 1] = 1.0
  e2 = jnp.asarray(e2_np)
  o2 = jnp.asarray(o2_np)
  et2 = jnp.asarray(e2_np.T.copy())
  ot2 = jnp.asarray(o2_np.T.copy())

  z1 = _mm1(degs_col, x_pad, W1, e2, o2)
  p1 = _propagate(z1.reshape(2 * N_PAD, DH), srcw, dstw)
  z2 = _mm2(degs_col, degd_col, p1.reshape(2, N_PAD // 2, 128), W2, b1r,
            e2, o2, et2, ot2)
  p2 = _propagate(z2.reshape(2 * N_PAD, DH), srcw, dstw)
  h = _fin(degd_col, p2.reshape(2, N_PAD // 2, 128), b2r, et2, ot2)
  return h[:N_NODES]


# speedup vs baseline: 1.0035x; 1.0035x over previous
import functools

import jax
import jax.numpy as jnp
import numpy as np
from jax import lax
from jax.experimental import pallas as pl
from jax.experimental.pallas import tpu as pltpu
from jax.experimental.pallas import tpu_sc as plsc

N_NODES = 10000
N_EDGES = 320000
D = 128
DH = 64

NC = 2
NS = 16
NW = NC * NS

N_PAD = 10240
ECHUNK = 128
CPT = 160
EPT = CPT * ECHUNK
E_PAD = NS * EPT
PAD_IDX = N_PAD - 1

NBUF = 4
LOOKAHEAD = 2
BLK = 8
RB = 3
RROWS = RB * BLK
NBLK = CPT // BLK
TROWS = NS * CPT
SLAB_ROWS = TROWS + 2 * BLK

ROWS_PER_TILE = N_PAD // NS
DEG_PER_TILE = E_PAD // NS


def _deg_body(srcw_hbm, dstw_hbm, deg_hbm, slab_v, deg2d_v, iota_v, zbuf_v, acc):
  c = lax.axis_index("c")
  s = lax.axis_index("s")

  @pl.when(c == 0)
  def _():
    pltpu.sync_copy(srcw_hbm.at[pl.ds(s * CPT, CPT)], slab_v)

  @pl.when(c == 1)
  def _():
    pltpu.sync_copy(dstw_hbm.at[pl.ds(s * CPT, CPT)], slab_v)

  for k in range(8):
    z = jnp.zeros((16,), jnp.float32)
    for r in range(8):
      zbuf_v[r, pl.ds(16 * k, 16)] = z

  @pl.when(s < 10)
  def _():
    pltpu.sync_copy(zbuf_v, acc.at[pl.ds(s * 8, 8)])

  @pl.loop(0, 80)
  def _(r):
    z16 = jnp.zeros((16,), jnp.float32)
    for k in range(8):
      deg2d_v[r, pl.ds(16 * k, 16)] = z16

  for r in range(5):
    iota_v[r, :] = lax.iota(jnp.int32, 16) + 16 * r
  plsc.subcore_barrier()

  ones = jnp.ones((16,), jnp.float32)

  @pl.loop(0, CPT)
  def _(r):
    for k in range(ECHUNK // 16):
      v = slab_v[r, pl.ds(16 * k, 16)]
      rows = lax.shift_right_logical(v, 7)
      cols = jnp.bitwise_and(v, 127)
      plsc.addupdate_scatter(deg2d_v, (rows, cols), ones)

  for r in range(5):
    pltpu.sync_copy(deg2d_v.at[pl.ds(r * 16, 16)], acc.at[iota_v.at[r]],
                    add=True)
  plsc.subcore_barrier()

  @pl.when(s < 10)
  def _():
    pltpu.sync_copy(acc.at[pl.ds(s * 8, 8)], deg_hbm.at[c, pl.ds(s * 8, 8)])


def _degrees(srcw, dstw):
  mesh = plsc.VectorSubcoreMesh(core_axis_name="c", subcore_axis_name="s",
                                num_cores=NC, num_subcores=NS)
  return pl.kernel(
      _deg_body,
      out_type=jax.ShapeDtypeStruct((2, 80, 128), jnp.float32),
      mesh=mesh,
      compiler_params=pltpu.CompilerParams(needs_layout_passes=False),
      scratch_types=[
          pltpu.VMEM((CPT, ECHUNK), jnp.int32),
          pltpu.VMEM((80, 128), jnp.float32),
          pltpu.VMEM((5, 16), jnp.int32),
          pltpu.VMEM((8, 128), jnp.float32),
          pltpu.VMEM_SHARED((80, 128), jnp.float32),
      ],
  )(srcw, dstw)


def _prop_body(z_hbm, srcw_hbm, dstw_hbm, p_hbm,
               ring_s, ring_d, *rest):
  bufs = rest[:NBUF]
  zbuf_v = rest[NBUF]
  gsems = rest[NBUF + 1:2 * NBUF + 1]
  ssems = rest[2 * NBUF + 1:3 * NBUF + 1]
  rsem = rest[3 * NBUF + 1]
  zsp = rest[3 * NBUF + 2]
  acc = rest[3 * NBUF + 3]

  c = lax.axis_index("c")
  s = lax.axis_index("s")

  pltpu.sync_copy(z_hbm.at[pl.ds(c * N_PAD + s * ROWS_PER_TILE, ROWS_PER_TILE)],
                  zsp.at[pl.ds(s * ROWS_PER_TILE, ROWS_PER_TILE)])

  for k in range(4):
    z = jnp.zeros((16,), jnp.float32)
    for r in range(16):
      zbuf_v[r, pl.ds(16 * k, 16)] = z

  @pl.loop(0, ROWS_PER_TILE // 16)
  def _(i):
    pltpu.async_copy(zbuf_v, acc.at[pl.ds(s * ROWS_PER_TILE + i * 16, 16)],
                     rsem)

  @pl.loop(0, ROWS_PER_TILE // 16)
  def _(i):
    pltpu.make_async_copy(
        zbuf_v, acc.at[pl.ds(s * ROWS_PER_TILE, 16)], rsem).wait()

  def slab_row(r0):
    if isinstance(r0, int):
      return s * CPT + r0 if r0 < CPT else TROWS + (r0 - CPT)
    return jnp.where(r0 < CPT, s * CPT + r0, TROWS + (r0 - CPT))

  def issue_ring(row0):
    rr = row0 % RROWS if isinstance(row0, int) else lax.rem(row0, RROWS)
    g0 = slab_row(row0)
    pltpu.async_copy(srcw_hbm.at[pl.ds(g0, BLK)],
                     ring_s.at[pl.ds(rr, BLK)], rsem)
    pltpu.async_copy(dstw_hbm.at[pl.ds(g0, BLK)],
                     ring_d.at[pl.ds(rr, BLK)], rsem)

  def wait_ring():
    pltpu.make_async_copy(srcw_hbm.at[pl.ds(0, BLK)],
                          ring_s.at[pl.ds(0, BLK)], rsem).wait()
    pltpu.make_async_copy(dstw_hbm.at[pl.ds(0, BLK)],
                          ring_d.at[pl.ds(0, BLK)], rsem).wait()

  def issue_gather(chunk, b):
    rr = chunk % RROWS if isinstance(chunk, int) else lax.rem(chunk, RROWS)
    pltpu.async_copy(zsp.at[ring_s.at[rr]], bufs[b], gsems[b])

  def wait_gather(b):
    pltpu.make_async_copy(zsp.at[ring_s.at[0]], bufs[b], gsems[b]).wait()

  def issue_scatter(chunk, b):
    rr = chunk % RROWS if isinstance(chunk, int) else lax.rem(chunk, RROWS)
    pltpu.async_copy(bufs[b], acc.at[ring_d.at[rr]], ssems[b], add=True)

  def wait_scatter(b):
    pltpu.make_async_copy(bufs[b], acc.at[ring_d.at[0]], ssems[b]).wait()

  for g in range(RB):
    issue_ring(g * BLK)
  wait_ring()
  wait_ring()

  plsc.subcore_barrier()

  for i in range(LOOKAHEAD):
    issue_gather(i, i % NBUF)
  for i in range(BLK):
    bf = (i + LOOKAHEAD) % NBUF
    if i + LOOKAHEAD >= NBUF:
      wait_scatter(bf)
    issue_gather(i + LOOKAHEAD, bf)
    wait_gather(i % NBUF)
    issue_scatter(i, i % NBUF)

  @pl.loop(BLK, CPT, step=BLK)
  def _(j):
    wait_ring()
    for p in range(BLK):
      i = j + p
      b = p % NBUF
      bf = (p + LOOKAHEAD) % NBUF
      wait_scatter(bf)
      issue_gather(i + LOOKAHEAD, bf)
      wait_gather(b)
      issue_scatter(i, b)
    issue_ring(j + 2 * BLK)

  for i in range(CPT - (NBUF - LOOKAHEAD), CPT):
    wait_scatter(i % NBUF)
  for i in range(CPT, CPT + LOOKAHEAD):
    wait_gather(i % NBUF)
  wait_ring()

  plsc.subcore_barrier()
  pltpu.sync_copy(acc.at[pl.ds(s * ROWS_PER_TILE, ROWS_PER_TILE)],
                  p_hbm.at[c, pl.ds(s * ROWS_PER_TILE, ROWS_PER_TILE)])


@functools.cache
def _propagate_kernel():
  mesh = plsc.VectorSubcoreMesh(core_axis_name="c", subcore_axis_name="s",
                                num_cores=NC, num_subcores=NS)
  return pl.kernel(
      _prop_body,
      out_type=jax.ShapeDtypeStruct((2, N_PAD, DH), jnp.float32),
      mesh=mesh,
      compiler_params=pltpu.CompilerParams(needs_layout_passes=False,
                                           use_tc_tiling_on_sc=False),
      scratch_types=(
          [pltpu.VMEM((RROWS, ECHUNK), jnp.int32),
           pltpu.VMEM((RROWS, ECHUNK), jnp.int32)]
          + [pltpu.VMEM((ECHUNK, DH), jnp.float32) for _ in range(NBUF)]
          + [pltpu.VMEM((16, DH), jnp.float32)]
          + [pltpu.SemaphoreType.DMA for _ in range(2 * NBUF + 1)]
          + [pltpu.VMEM_SHARED((N_PAD, DH), jnp.float32),
             pltpu.VMEM_SHARED((N_PAD, DH), jnp.float32)]
      ),
  )


@jax.jit
def _propagate(zflat, srcw, dstw):
  return _propagate_kernel()(zflat, srcw, dstw)


def _norm_col(deg_ref):
  return lax.rsqrt(jnp.maximum(deg_ref[...], 1.0))


def _emit(e2, o2, th):
  return jnp.concatenate(
      [jnp.dot(e2, th, preferred_element_type=jnp.float32),
       jnp.dot(o2, th, preferred_element_type=jnp.float32)], axis=1)


def _deinter(et2, ot2, q):
  return (jnp.dot(et2, q[:, :64], preferred_element_type=jnp.float32)
          + jnp.dot(ot2, q[:, 64:], preferred_element_type=jnp.float32))


def _mm1_body(degs_ref, x_ref, w_ref, e2_ref, o2_ref, o_ref):
  t = jnp.dot(x_ref[...], w_ref[...], preferred_element_type=jnp.float32)
  ts = t * _norm_col(degs_ref)
  e2, o2 = e2_ref[...], o2_ref[...]
  q0 = _emit(e2, o2, ts[:, :64])
  q1 = _emit(e2, o2, ts[:, 64:])
  o_ref[...] = jnp.concatenate([q0[None], q1[None]], axis=0)


def _mm1(degs_col, x_pad, w1, e2, o2):
  return pl.pallas_call(
      _mm1_body,
      grid=(20,),
      in_specs=[
          pl.BlockSpec((512, 1), lambda i: (i, 0)),
          pl.BlockSpec((512, 128), lambda i: (i, 0)),
          pl.BlockSpec((128, 128), lambda i: (0, 0)),
          pl.BlockSpec((256, 512), lambda i: (0, 0)),
          pl.BlockSpec((256, 512), lambda i: (0, 0)),
      ],
      out_specs=pl.BlockSpec((2, 256, 128), lambda i: (0, i, 0)),
      out_shape=jax.ShapeDtypeStruct((2, N_PAD // 2, 128), jnp.float32),
  )(degs_col, x_pad, w1, e2, o2)


def _mm2_body(degs_ref, degd_ref, p_ref, w_ref, b_ref, e2_ref, o2_ref,
              et2_ref, ot2_ref, o_ref):
  et2, ot2 = et2_ref[...], ot2_ref[...]
  cat = jnp.concatenate([_deinter(et2, ot2, p_ref[0]),
                         _deinter(et2, ot2, p_ref[1])], axis=1)
  h = cat * _norm_col(degd_ref) + b_ref[...]
  t = jnp.dot(h, w_ref[...], preferred_element_type=jnp.float32)
  ts = t * _norm_col(degs_ref)
  e2, o2 = e2_ref[...], o2_ref[...]
  q0 = _emit(e2, o2, ts[:, :64])
  q1 = _emit(e2, o2, ts[:, 64:])
  o_ref[...] = jnp.concatenate([q0[None], q1[None]], axis=0)


def _mm2(degs_col, degd_col, p, w2, b1, e2, o2, et2, ot2):
  return pl.pallas_call(
      _mm2_body,
      grid=(20,),
      in_specs=[
          pl.BlockSpec((512, 1), lambda i: (i, 0)),
          pl.BlockSpec((512, 1), lambda i: (i, 0)),
          pl.BlockSpec((2, 256, 128), lambda i: (0, i, 0)),
          pl.BlockSpec((128, 128), lambda i: (0, 0)),
          pl.BlockSpec((1, 128), lambda i: (0, 0)),
          pl.BlockSpec((256, 512), lambda i: (0, 0)),
          pl.BlockSpec((256, 512), lambda i: (0, 0)),
          pl.BlockSpec((512, 256), lambda i: (0, 0)),
          pl.BlockSpec((512, 256), lambda i: (0, 0)),
      ],
      out_specs=pl.BlockSpec((2, 256, 128), lambda i: (0, i, 0)),
      out_shape=jax.ShapeDtypeStruct((2, N_PAD // 2, 128), jnp.float32),
  )(degs_col, degd_col, p, w2, b1, e2, o2, et2, ot2)


def _fin_body(degd_ref, p_ref, b_ref, et2_ref, ot2_ref, o_ref):
  et2, ot2 = et2_ref[...], ot2_ref[...]
  cat = jnp.concatenate([_deinter(et2, ot2, p_ref[0]),
                         _deinter(et2, ot2, p_ref[1])], axis=1)
  o_ref[...] = cat * _norm_col(degd_ref) + b_ref[...]


def _fin(degd_col, p, b2, et2, ot2):
  return pl.pallas_call(
      _fin_body,
      grid=(20,),
      in_specs=[
          pl.BlockSpec((512, 1), lambda i: (i, 0)),
          pl.BlockSpec((2, 256, 128), lambda i: (0, i, 0)),
          pl.BlockSpec((1, 128), lambda i: (0, 0)),
          pl.BlockSpec((512, 256), lambda i: (0, 0)),
          pl.BlockSpec((512, 256), lambda i: (0, 0)),
      ],
      out_specs=pl.BlockSpec((512, 128), lambda i: (i, 0)),
      out_shape=jax.ShapeDtypeStruct((N_PAD, D), jnp.float32),
  )(degd_col, p, b2, et2, ot2)


@jax.jit
def kernel(x, edge_index, W1, b1, W2, b2):
  src = edge_index[0].astype(jnp.int32)
  dst = edge_index[1].astype(jnp.int32)
  pad = jnp.full((SLAB_ROWS * ECHUNK - N_EDGES,), PAD_IDX, jnp.int32)
  srcw = jnp.concatenate([src, pad]).reshape(SLAB_ROWS, ECHUNK)
  dstw = jnp.concatenate([dst, pad]).reshape(SLAB_ROWS, ECHUNK)

  deg = _degrees(srcw, dstw)
  degs_col = deg[0].reshape(N_PAD, 1)
  degd_col = deg[1].reshape(N_PAD, 1)

  x_pad = jnp.pad(x, ((0, N_PAD - N_NODES), (0, 0)))
  b1r = b1.reshape(1, D)
  b2r = b2.reshape(1, D)

  ii = np.arange(256)
  e2_np = np.zeros((256, 512), np.float32)
  o2_np = np.zeros((256, 512), np.float32)
  e2_np[ii, 2 * ii] = 1.0
  o2_np[ii, 2 * ii + 1] = 1.0
  e2 = jnp.asarray(e2_np)
  o2 = jnp.asarray(o2_np)
  et2 = jnp.asarray(e2_np.T.copy())
  ot2 = jnp.asarray(o2_np.T.copy())

  z1 = _mm1(degs_col, x_pad, W1, e2, o2)
  p1 = _propagate(z1.reshape(2 * N_PAD, DH), srcw, dstw)
  z2 = _mm2(degs_col, degd_col, p1.reshape(2, N_PAD // 2, 128), W2, b1r,
            e2, o2, et2, ot2)
  p2 = _propagate(z2.reshape(2 * N_PAD, DH), srcw, dstw)
  h = _fin(degd_col, p2.reshape(2, N_PAD // 2, 128), b2r, et2, ot2)
  return h[:N_NODES]
